# baseline (speedup 1.0000x reference)
import functools

import jax
import jax.numpy as jnp
from jax import lax
from jax.experimental import pallas as pl
from jax.experimental.pallas import tpu as pltpu
from jax.experimental.pallas import tpu_sc as plsc

_NC = 2
_NS = 16
_NW = _NC * _NS
_L = 16

_BATCH = 16384
_D = 64
_N = 1000000
_NMAIN = 999936
_ROWS = 500032
_W = 128

_SCOL = 384
_NSTRIPE = _NMAIN // _SCOL
_SPW = (_NSTRIPE + _NW - 1) // _NW
_YLEN = _SCOL // 2 * _W
_XB = _D * _SCOL * 4
_YB = _YLEN * 4
_TAIL_OFF = _NMAIN // 2 * _W

_BPW = _BATCH // _NW
_CHUNK = 256
_NCHUNK = _BPW // _CHUNK
_GROUPS = _CHUNK // _L


def _tr_body(pt_hbm, tail_hbm, out_hbm, x0, x1, y0, y1,
             si0, si1, so0, so1):
    wid = lax.axis_index("s") * _NC + lax.axis_index("c")
    col64 = lax.iota(jnp.int32, _L) * _D
    bufs = ((x0, y0, si0, so0), (x1, y1, si1, so1))

    def stripe(j):
        return jnp.minimum(wid + j * _NW, _NSTRIPE - 1)

    def fire_in(j, xb, s):
        c0 = pl.multiple_of(stripe(j) * _SCOL, _SCOL)
        pltpu.async_copy(pt_hbm.at[:, pl.ds(c0, _SCOL)], xb, s)

    def wait_in(xb, s):
        pltpu.make_async_copy(pt_hbm.at[:, pl.ds(0, _SCOL)], xb, s).wait()

    def scatter(xb, yb):
        def f_loop(f, carry):
            ib = col64 + f
            for c2 in range(_SCOL // _L):
                v = xb[f, pl.ds(c2 * _L, _L)]
                plsc.store_scatter(yb, [ib + c2 * 1024], v)
            return carry
        lax.fori_loop(0, _D, f_loop, 0)

    def fire_out(j, yb, s):
        o0 = pl.multiple_of(stripe(j) * _YLEN, _YLEN)
        pltpu.async_copy(yb, out_hbm.at[pl.ds(o0, _YLEN)], s)

    def wait_out(yb, s):
        pltpu.make_async_copy(out_hbm.at[pl.ds(0, _YLEN)], yb, s).wait()

    fire_in(0, x0, si0)
    fire_in(1, x1, si1)

    def do_pair(k, first):
        for slot in (0, 1):
            xb, yb, si, so = bufs[slot]
            j = 2 * k + slot
            wait_in(xb, si)
            if not first:
                wait_out(yb, so)
            scatter(xb, yb)
            fire_in(j + 2, xb, si)
            fire_out(j, yb, so)

    do_pair(0, True)
    lax.fori_loop(1, _SPW // 2, lambda k, c: (do_pair(k, False), c)[1], 0)

    for xb, yb, si, so in bufs:
        wait_in(xb, si)
        wait_out(yb, so)

    @pl.when(wid == _NW - 1)
    def _tail():
        pltpu.sync_copy(tail_hbm, x0.at[:, pl.ds(0, _W)])

        def f_loop(f, carry):
            ib = col64 + f
            for c2 in range(_W // _L):
                v = x0[f, pl.ds(c2 * _L, _L)]
                plsc.store_scatter(y0, [ib + c2 * 1024], v)
            return carry
        lax.fori_loop(0, _D, f_loop, 0)
        pltpu.sync_copy(y0.at[pl.ds(0, _W // 2 * _W)],
                        out_hbm.at[pl.ds(_TAIL_OFF, _W // 2 * _W)])


def _transpose_table(T):
    pt = T.T
    tail = jnp.pad(T[_NMAIN:, :].T, ((0, 0), (0, _W - _D)))
    mesh = plsc.VectorSubcoreMesh(core_axis_name="c", subcore_axis_name="s")
    run = functools.partial(
        pl.kernel,
        mesh=mesh,
        compiler_params=pltpu.CompilerParams(
            needs_layout_passes=False, use_tc_tiling_on_sc=True),
        out_type=jax.ShapeDtypeStruct((_ROWS * _W,), jnp.float32),
        scratch_types=[
            pltpu.VMEM((_D, _SCOL), jnp.float32),
            pltpu.VMEM((_D, _SCOL), jnp.float32),
            pltpu.VMEM((_YLEN,), jnp.float32),
            pltpu.VMEM((_YLEN,), jnp.float32),
            pltpu.SemaphoreType.DMA,
            pltpu.SemaphoreType.DMA,
            pltpu.SemaphoreType.DMA,
            pltpu.SemaphoreType.DMA,
        ],
    )(_tr_body)
    return run(pt, tail).reshape(_ROWS, _W)


def _sc_body(u_hbm, i_hbm, mu_hbm, bu_hbm, bi_hbm, p_hbm, q_hbm, out_hbm,
             uidx_v, iidx_v, u2_v, i2_v, pu_v, qi_v, bu_v, bi_v, mu_v,
             out_v, sem):
    wid = lax.axis_index("s") * _NC + lax.axis_index("c")
    base = wid * _BPW
    pltpu.sync_copy(mu_hbm, mu_v)
    mu_vec = mu_v[...]

    lane_iota = lax.iota(jnp.int32, _L)
    lane_masks = [lane_iota == r for r in range(_L)]

    for chunk in range(_NCHUNK):
        cbase = base + chunk * _CHUNK
        pltpu.sync_copy(u_hbm.at[pl.ds(cbase, _CHUNK)], uidx_v)
        pltpu.sync_copy(i_hbm.at[pl.ds(cbase, _CHUNK)], iidx_v)
        bias_cps = [
            pltpu.async_copy(bu_hbm.at[uidx_v], bu_v, sem),
            pltpu.async_copy(bi_hbm.at[iidx_v], bi_v, sem),
        ]

        def halve(k, carry):
            sl = pl.ds(pl.multiple_of(k * _L, _L), _L)
            u2_v[sl] = lax.shift_right_logical(uidx_v[sl], 1)
            i2_v[sl] = lax.shift_right_logical(iidx_v[sl], 1)
            return carry

        lax.fori_loop(0, _CHUNK // _L, halve, 0)

        cps = [
            pltpu.async_copy(p_hbm.at[u2_v], pu_v, sem),
            pltpu.async_copy(q_hbm.at[i2_v], qi_v, sem),
        ]
        for cp in cps + bias_cps:
            cp.wait()

        def group(g, carry):
            gb = pl.multiple_of(g * _L, _L)
            sl = pl.ds(gb, _L)
            d_ee = jnp.zeros((_L,), jnp.float32)
            d_eo = jnp.zeros((_L,), jnp.float32)
            d_oe = jnp.zeros((_L,), jnp.float32)
            d_oo = jnp.zeros((_L,), jnp.float32)
            for r in range(_L):
                b = gb + r
                a_ee = jnp.zeros((_L,), jnp.float32)
                a_eo = jnp.zeros((_L,), jnp.float32)
                a_oe = jnp.zeros((_L,), jnp.float32)
                a_oo = jnp.zeros((_L,), jnp.float32)
                for c in range(_D // _L):
                    pe = pu_v[b, pl.ds(c * _L, _L)]
                    po = pu_v[b, pl.ds(_D + c * _L, _L)]
                    qe = qi_v[b, pl.ds(c * _L, _L)]
                    qo = qi_v[b, pl.ds(_D + c * _L, _L)]
                    a_ee = a_ee + pe * qe
                    a_eo = a_eo + pe * qo
                    a_oe = a_oe + po * qe
                    a_oo = a_oo + po * qo
                m = lane_masks[r]
                d_ee = jnp.where(m, jnp.sum(a_ee), d_ee)
                d_eo = jnp.where(m, jnp.sum(a_eo), d_eo)
                d_oe = jnp.where(m, jnp.sum(a_oe), d_oe)
                d_oo = jnp.where(m, jnp.sum(a_oo), d_oo)
            u_even = (uidx_v[sl] & 1) == 0
            i_even = (iidx_v[sl] & 1) == 0
            dots = jnp.where(
                u_even,
                jnp.where(i_even, d_ee, d_eo),
                jnp.where(i_even, d_oe, d_oo))
            out_v[sl] = mu_vec + bu_v[sl] + bi_v[sl] + dots
            return carry

        lax.fori_loop(0, _GROUPS, group, 0)
        pltpu.sync_copy(out_v, out_hbm.at[pl.ds(cbase, _CHUNK)])


def kernel(u_idx, i_idx, mu, b_u, b_i, P, Q):
    u_idx = u_idx.astype(jnp.int32)
    i_idx = i_idx.astype(jnp.int32)
    mu_vec = jnp.broadcast_to(mu.astype(jnp.float32), (_L,))
    sp = _transpose_table(P)
    sq = _transpose_table(Q)
    mesh = plsc.VectorSubcoreMesh(core_axis_name="c", subcore_axis_name="s")
    run = functools.partial(
        pl.kernel,
        mesh=mesh,
        compiler_params=pltpu.CompilerParams(needs_layout_passes=False),
        out_type=jax.ShapeDtypeStruct((_BATCH,), jnp.float32),
        scratch_types=[
            pltpu.VMEM((_CHUNK,), jnp.int32),
            pltpu.VMEM((_CHUNK,), jnp.int32),
            pltpu.VMEM((_CHUNK,), jnp.int32),
            pltpu.VMEM((_CHUNK,), jnp.int32),
            pltpu.VMEM((_CHUNK, _W), jnp.float32),
            pltpu.VMEM((_CHUNK, _W), jnp.float32),
            pltpu.VMEM((_CHUNK,), jnp.float32),
            pltpu.VMEM((_CHUNK,), jnp.float32),
            pltpu.VMEM((_L,), jnp.float32),
            pltpu.VMEM((_CHUNK,), jnp.float32),
            pltpu.SemaphoreType.DMA,
        ],
    )(_sc_body)
    return run(u_idx, i_idx, mu_vec, b_u, b_i, sp, sq)


# speedup vs baseline: 2.3085x; 2.3085x over previous
import functools

import jax
import jax.numpy as jnp
from jax import lax
from jax.experimental import pallas as pl
from jax.experimental.pallas import tpu as pltpu
from jax.experimental.pallas import tpu_sc as plsc

_NC = 2
_NS = 16
_NW = _NC * _NS
_L = 16

_BATCH = 16384
_D = 64
_N = 1000000
_NMAIN = 999936
_ROWS = 500032
_W = 128

_SCOL = 384
_NSTRIPE = _NMAIN // _SCOL
_SPW = (_NSTRIPE + _NW - 1) // _NW
_YLEN = _SCOL // 2 * _W
_XB = _D * _SCOL * 4
_YB = _YLEN * 4
_TAIL_OFF = _NMAIN // 2 * _W

_BPW = _BATCH // _NW
_CHUNK = 256
_NCHUNK = _BPW // _CHUNK
_GROUPS = _CHUNK // _L


def _tr_body(pt_hbm, tail_hbm, out_hbm, x0, x1, y0, y1,
             si0, si1, so0, so1):
    wid = lax.axis_index("s") * _NC + lax.axis_index("c")
    lane = lax.iota(jnp.int32, _L)
    perm = [(lane + d) & (_L - 1) for d in range(_L)]
    ev = [_D * perm[d] + lane for d in range(_L)]
    rows = [lane + 16 * a for a in range(_D // _L)]
    bufs = ((x0, y0, si0, so0), (x1, y1, si1, so1))

    def stripe(j):
        return jnp.minimum(wid + j * _NW, _NSTRIPE - 1)

    def fire_in(j, xb, s):
        c0 = pl.multiple_of(stripe(j) * _SCOL, _SCOL)
        pltpu.async_copy(pt_hbm.at[:, pl.ds(c0, _SCOL)], xb, s)

    def wait_in(xb, s):
        pltpu.make_async_copy(pt_hbm.at[:, pl.ds(0, _SCOL)], xb, s).wait()

    def scatter(xb, yb, nb):
        def b_loop(b, carry):
            uc0 = b * _L
            for a in range(_D // _L):
                base = b * 1024 + a * _L
                for d in range(_L):
                    ci = perm[d] + uc0
                    v = plsc.load_gather(xb, [rows[a], ci])
                    plsc.store_scatter(yb, [ev[d] + base], v)
            return carry
        lax.fori_loop(0, nb, b_loop, 0)

    def fire_out(j, yb, s):
        o0 = pl.multiple_of(stripe(j) * _YLEN, _YLEN)
        pltpu.async_copy(yb, out_hbm.at[pl.ds(o0, _YLEN)], s)

    def wait_out(yb, s):
        pltpu.make_async_copy(out_hbm.at[pl.ds(0, _YLEN)], yb, s).wait()

    fire_in(0, x0, si0)
    fire_in(1, x1, si1)

    def do_pair(k, first):
        for slot in (0, 1):
            xb, yb, si, so = bufs[slot]
            j = 2 * k + slot
            wait_in(xb, si)
            if not first:
                wait_out(yb, so)
            scatter(xb, yb, _SCOL // _L)
            fire_in(j + 2, xb, si)
            fire_out(j, yb, so)

    do_pair(0, True)
    lax.fori_loop(1, _SPW // 2, lambda k, c: (do_pair(k, False), c)[1], 0)

    for xb, yb, si, so in bufs:
        wait_in(xb, si)
        wait_out(yb, so)

    @pl.when(wid == _NW - 1)
    def _tail():
        pltpu.sync_copy(tail_hbm, x0.at[:, pl.ds(0, _W)])
        scatter(x0, y0, _W // _L)
        pltpu.sync_copy(y0.at[pl.ds(0, _W // 2 * _W)],
                        out_hbm.at[pl.ds(_TAIL_OFF, _W // 2 * _W)])


def _transpose_table(T):
    pt = T.T
    tail = jnp.pad(T[_NMAIN:, :].T, ((0, 0), (0, _W - _D)))
    mesh = plsc.VectorSubcoreMesh(core_axis_name="c", subcore_axis_name="s")
    run = functools.partial(
        pl.kernel,
        mesh=mesh,
        compiler_params=pltpu.CompilerParams(
            needs_layout_passes=False, use_tc_tiling_on_sc=True),
        out_type=jax.ShapeDtypeStruct((_ROWS * _W,), jnp.float32),
        scratch_types=[
            pltpu.VMEM((_D, _SCOL), jnp.float32),
            pltpu.VMEM((_D, _SCOL), jnp.float32),
            pltpu.VMEM((_YLEN,), jnp.float32),
            pltpu.VMEM((_YLEN,), jnp.float32),
            pltpu.SemaphoreType.DMA,
            pltpu.SemaphoreType.DMA,
            pltpu.SemaphoreType.DMA,
            pltpu.SemaphoreType.DMA,
        ],
    )(_tr_body)
    return run(pt, tail).reshape(_ROWS, _W)


def _sc_body(u_hbm, i_hbm, mu_hbm, bu_hbm, bi_hbm, p_hbm, q_hbm, out_hbm,
             uidx_v, iidx_v, u2_v, i2_v, pu_v, qi_v, bu_v, bi_v, mu_v,
             out_v, sem):
    wid = lax.axis_index("s") * _NC + lax.axis_index("c")
    base = wid * _BPW
    pltpu.sync_copy(mu_hbm, mu_v)
    mu_vec = mu_v[...]

    lane_iota = lax.iota(jnp.int32, _L)
    lane_masks = [lane_iota == r for r in range(_L)]

    for chunk in range(_NCHUNK):
        cbase = base + chunk * _CHUNK
        pltpu.sync_copy(u_hbm.at[pl.ds(cbase, _CHUNK)], uidx_v)
        pltpu.sync_copy(i_hbm.at[pl.ds(cbase, _CHUNK)], iidx_v)
        bias_cps = [
            pltpu.async_copy(bu_hbm.at[uidx_v], bu_v, sem),
            pltpu.async_copy(bi_hbm.at[iidx_v], bi_v, sem),
        ]

        def halve(k, carry):
            sl = pl.ds(pl.multiple_of(k * _L, _L), _L)
            u2_v[sl] = lax.shift_right_logical(uidx_v[sl], 1)
            i2_v[sl] = lax.shift_right_logical(iidx_v[sl], 1)
            return carry

        lax.fori_loop(0, _CHUNK // _L, halve, 0)

        cps = [
            pltpu.async_copy(p_hbm.at[u2_v], pu_v, sem),
            pltpu.async_copy(q_hbm.at[i2_v], qi_v, sem),
        ]
        for cp in cps + bias_cps:
            cp.wait()

        def group(g, carry):
            gb = pl.multiple_of(g * _L, _L)
            sl = pl.ds(gb, _L)
            d_ee = jnp.zeros((_L,), jnp.float32)
            d_eo = jnp.zeros((_L,), jnp.float32)
            d_oe = jnp.zeros((_L,), jnp.float32)
            d_oo = jnp.zeros((_L,), jnp.float32)
            for r in range(_L):
                b = gb + r
                a_ee = jnp.zeros((_L,), jnp.float32)
                a_eo = jnp.zeros((_L,), jnp.float32)
                a_oe = jnp.zeros((_L,), jnp.float32)
                a_oo = jnp.zeros((_L,), jnp.float32)
                for c in range(_D // _L):
                    pe = pu_v[b, pl.ds(c * _L, _L)]
                    po = pu_v[b, pl.ds(_D + c * _L, _L)]
                    qe = qi_v[b, pl.ds(c * _L, _L)]
                    qo = qi_v[b, pl.ds(_D + c * _L, _L)]
                    a_ee = a_ee + pe * qe
                    a_eo = a_eo + pe * qo
                    a_oe = a_oe + po * qe
                    a_oo = a_oo + po * qo
                m = lane_masks[r]
                d_ee = jnp.where(m, jnp.sum(a_ee), d_ee)
                d_eo = jnp.where(m, jnp.sum(a_eo), d_eo)
                d_oe = jnp.where(m, jnp.sum(a_oe), d_oe)
                d_oo = jnp.where(m, jnp.sum(a_oo), d_oo)
            u_even = (uidx_v[sl] & 1) == 0
            i_even = (iidx_v[sl] & 1) == 0
            dots = jnp.where(
                u_even,
                jnp.where(i_even, d_ee, d_eo),
                jnp.where(i_even, d_oe, d_oo))
            out_v[sl] = mu_vec + bu_v[sl] + bi_v[sl] + dots
            return carry

        lax.fori_loop(0, _GROUPS, group, 0)
        pltpu.sync_copy(out_v, out_hbm.at[pl.ds(cbase, _CHUNK)])


def kernel(u_idx, i_idx, mu, b_u, b_i, P, Q):
    u_idx = u_idx.astype(jnp.int32)
    i_idx = i_idx.astype(jnp.int32)
    mu_vec = jnp.broadcast_to(mu.astype(jnp.float32), (_L,))
    sp = _transpose_table(P)
    sq = _transpose_table(Q)
    mesh = plsc.VectorSubcoreMesh(core_axis_name="c", subcore_axis_name="s")
    run = functools.partial(
        pl.kernel,
        mesh=mesh,
        compiler_params=pltpu.CompilerParams(needs_layout_passes=False),
        out_type=jax.ShapeDtypeStruct((_BATCH,), jnp.float32),
        scratch_types=[
            pltpu.VMEM((_CHUNK,), jnp.int32),
            pltpu.VMEM((_CHUNK,), jnp.int32),
            pltpu.VMEM((_CHUNK,), jnp.int32),
            pltpu.VMEM((_CHUNK,), jnp.int32),
            pltpu.VMEM((_CHUNK, _W), jnp.float32),
            pltpu.VMEM((_CHUNK, _W), jnp.float32),
            pltpu.VMEM((_CHUNK,), jnp.float32),
            pltpu.VMEM((_CHUNK,), jnp.float32),
            pltpu.VMEM((_L,), jnp.float32),
            pltpu.VMEM((_CHUNK,), jnp.float32),
            pltpu.SemaphoreType.DMA,
        ],
    )(_sc_body)
    return run(u_idx, i_idx, mu_vec, b_u, b_i, sp, sq)


# speedup vs baseline: 2.7180x; 1.1774x over previous
import functools

import jax
import jax.numpy as jnp
from jax import lax
from jax.experimental import pallas as pl
from jax.experimental.pallas import tpu as pltpu
from jax.experimental.pallas import tpu_sc as plsc

_NC = 2
_NS = 16
_NW = _NC * _NS
_L = 16

_BATCH = 16384
_D = 64
_W = 128
_BPW = _BATCH // _NW
_CHUNK = 256
_NCHUNK = _BPW // _CHUNK
_GROUPS = _CHUNK // _L


def _sc_body(u_hbm, i_hbm, mu_hbm, bu_hbm, bi_hbm, pq_hbm, out_hbm,
             uidx_v, iidx_v, pu_v, qi_v, bu_v, bi_v, mu_v, out_v, sem):
    wid = lax.axis_index("s") * _NC + lax.axis_index("c")
    base = wid * _BPW
    pltpu.sync_copy(mu_hbm, mu_v)
    mu_vec = mu_v[...]

    lane_iota = lax.iota(jnp.int32, _L)
    lane_masks = [lane_iota == r for r in range(_L)]

    for chunk in range(_NCHUNK):
        cbase = base + chunk * _CHUNK
        csl = pl.ds(cbase, _CHUNK)
        pltpu.sync_copy(u_hbm.at[csl], uidx_v)
        pltpu.sync_copy(i_hbm.at[csl], iidx_v)
        cps = [
            pltpu.async_copy(bu_hbm.at[uidx_v], bu_v, sem),
            pltpu.async_copy(bi_hbm.at[iidx_v], bi_v, sem),
            pltpu.async_copy(pq_hbm.at[uidx_v], pu_v, sem),
            pltpu.async_copy(pq_hbm.at[iidx_v], qi_v, sem),
        ]
        for cp in cps:
            cp.wait()

        def group(g, carry):
            gb = pl.multiple_of(g * _L, _L)
            sl = pl.ds(gb, _L)
            dots = jnp.zeros((_L,), jnp.float32)
            for r in range(_L):
                b = gb + r
                acc = pu_v[b, pl.ds(0, _L)] * qi_v[b, pl.ds(_D, _L)]
                for c in range(1, _D // _L):
                    acc = acc + (pu_v[b, pl.ds(c * _L, _L)] *
                                 qi_v[b, pl.ds(_D + c * _L, _L)])
                dots = jnp.where(lane_masks[r], jnp.sum(acc), dots)
            out_v[sl] = mu_vec + bu_v[sl] + bi_v[sl] + dots
            return carry

        lax.fori_loop(0, _GROUPS, group, 0)
        pltpu.sync_copy(out_v, out_hbm.at[csl])


def kernel(u_idx, i_idx, mu, b_u, b_i, P, Q):
    u_idx = u_idx.astype(jnp.int32)
    i_idx = i_idx.astype(jnp.int32)
    mu_vec = jnp.broadcast_to(mu.astype(jnp.float32), (_L,))
    PQ = jnp.concatenate([P, Q], axis=1)
    mesh = plsc.VectorSubcoreMesh(core_axis_name="c", subcore_axis_name="s")
    run = functools.partial(
        pl.kernel,
        mesh=mesh,
        compiler_params=pltpu.CompilerParams(
            needs_layout_passes=False, use_tc_tiling_on_sc=True),
        out_type=jax.ShapeDtypeStruct((_BATCH,), jnp.float32),
        scratch_types=[
            pltpu.VMEM((_CHUNK,), jnp.int32),
            pltpu.VMEM((_CHUNK,), jnp.int32),
            pltpu.VMEM((_CHUNK, _W), jnp.float32),
            pltpu.VMEM((_CHUNK, _W), jnp.float32),
            pltpu.VMEM((_CHUNK,), jnp.float32),
            pltpu.VMEM((_CHUNK,), jnp.float32),
            pltpu.VMEM((_L,), jnp.float32),
            pltpu.VMEM((_CHUNK,), jnp.float32),
            pltpu.SemaphoreType.DMA,
        ],
    )(_sc_body)
    return run(u_idx, i_idx, mu_vec, b_u, b_i, PQ)
